# baseline (speedup 1.0000x reference)
import jax
import jax.numpy as jnp
from jax.experimental import pallas as pl

_SBLK = 1024


def _add_body(x_ref, t_ref, o_ref):
    o_ref[...] = x_ref[...] + t_ref[...][None, :, :]


def kernel(inputs, pos_table):
    batch, seq, dim = inputs.shape
    return pl.pallas_call(
        _add_body,
        grid=(seq // _SBLK, batch),
        in_specs=[
            pl.BlockSpec((1, _SBLK, dim), lambda s, b: (b, s, 0)),
            pl.BlockSpec((_SBLK, dim), lambda s, b: (s, 0)),
        ],
        out_specs=pl.BlockSpec((1, _SBLK, dim), lambda s, b: (b, s, 0)),
        out_shape=jax.ShapeDtypeStruct((batch, seq, dim), jnp.float32),
    )(inputs, pos_table)


# speedup vs baseline: 1.0432x; 1.0432x over previous
import jax
import jax.numpy as jnp
from jax.experimental import pallas as pl

_SBLK = 2048


def _add_body(x_ref, t_ref, o_ref):
    o_ref[...] = x_ref[...] + t_ref[...][None, :, :]


def kernel(inputs, pos_table):
    batch, seq, dim = inputs.shape
    return pl.pallas_call(
        _add_body,
        grid=(seq // _SBLK, batch),
        in_specs=[
            pl.BlockSpec((1, _SBLK, dim), lambda s, b: (b, s, 0)),
            pl.BlockSpec((_SBLK, dim), lambda s, b: (s, 0)),
        ],
        out_specs=pl.BlockSpec((1, _SBLK, dim), lambda s, b: (b, s, 0)),
        out_shape=jax.ShapeDtypeStruct((batch, seq, dim), jnp.float32),
    )(inputs, pos_table)
